# baseline (speedup 1.0000x reference)
import functools

import jax
import jax.numpy as jnp
from jax import lax
from jax.experimental import pallas as pl
from jax.experimental.pallas import tpu as pltpu
from jax.experimental.pallas import tpu_sc as plsc

_NC = 2
_NS = 16
_NW = _NC * _NS
_L = 16
_NPH = 2


def _base_kernel_body(x_ref, c_ref, u_ref, o_ref):
    acc = lax.dot_general(x_ref[...], c_ref[...], (((1,), (1,)), ((), ())),
                          preferred_element_type=jnp.float32)
    o_ref[...] = -2.0 * acc + u_ref[...]


def _make_base(N, M, K, D, BN):
    nb = N // BN
    return pl.pallas_call(
        _base_kernel_body,
        grid=(M, nb),
        in_specs=[
            pl.BlockSpec((BN, D), lambda i, b: (b, 0)),
            pl.BlockSpec((K, D), lambda i, b: (i, 0)),
            pl.BlockSpec((BN, K), lambda i, b: (b, i)),
        ],
        out_specs=pl.BlockSpec((BN, K), lambda i, b, nb=nb: (i * nb + b, 0)),
        out_shape=jax.ShapeDtypeStruct((M * N, K), jnp.float32),
    )


def _make_transpose(MM, K):
    KP = K // _NPH

    def body(i_ref, *o_refs):
        x = i_ref[0]
        for p in range(_NPH):
            o_refs[p][...] = x[p * KP:(p + 1) * KP, :].T

    return pl.pallas_call(
        body,
        grid=(MM,),
        in_specs=[pl.BlockSpec((1, K, K), lambda g: (g, 0, 0))],
        out_specs=[pl.BlockSpec((K, KP), lambda g: (g, 0))
                   for _ in range(_NPH)],
        out_shape=[jax.ShapeDtypeStruct((MM * K, KP), jnp.float32)
                   for _ in range(_NPH)],
    )


def _make_icm(N, M, K, n_iter, rows_per_w):
    KP = K // _NPH
    SGP = KP // _L
    GR = _L
    ngrp = rows_per_w // GR
    nidx = GR * M
    nsweep = n_iter * M
    mesh = plsc.VectorSubcoreMesh(core_axis_name="c", subcore_axis_name="s")

    @functools.partial(
        pl.kernel,
        mesh=mesh,
        out_type=jax.ShapeDtypeStruct((M * N,), jnp.int32),
        scratch_types=[
            pltpu.VMEM((M * rows_per_w,), jnp.int32),
            [pltpu.VMEM((nidx,), jnp.int32) for _ in range(2)],
            [pltpu.VMEM((nidx, KP), jnp.float32) for _ in range(_NPH)],
            pltpu.VMEM((GR, K), jnp.float32),
            pltpu.VMEM((GR, _L), jnp.float32),
            pltpu.VMEM((GR, _L), jnp.int32),
            [pltpu.SemaphoreType.DMA for _ in range(_NPH)],
            pltpu.SemaphoreType.DMA,
        ],
    )
    def icm(base_hbm, t0, t1, b_hbm, out_hbm,
            b_loc, idx2, gb, bbuf, mv_s, mi_s, gsem, bsem):
        tabs = (t0, t1)
        wid = lax.axis_index("s") * _NC + lax.axis_index("c")
        n0 = wid * rows_per_w
        iota = lax.iota(jnp.int32, _L)
        inf16 = jnp.broadcast_to(jnp.float32(jnp.inf), (_L,))
        zero16 = jnp.broadcast_to(jnp.int32(0), (_L,))

        for j in range(M):
            pltpu.sync_copy(b_hbm.at[pl.ds(j * N + n0, rows_per_w)],
                            b_loc.at[pl.ds(j * rows_per_w, rows_per_w)])

        def build_idx(buf, row0, toff):
            for j in range(M):
                bv = b_loc[pl.ds(j * rows_per_w + row0, _L)]
                buf[pl.ds(j * _L, _L)] = bv + (toff + j * K)

        def issue_g(p, buf):
            pltpu.async_copy(tabs[p].at[buf], gb[p], gsem[p])

        def issue_b(row0, i_nxt):
            pltpu.async_copy(
                base_hbm.at[pl.ds(i_nxt * N + n0 + row0, GR)], bbuf, bsem)

        def wait_g(p):
            pltpu.make_async_copy(tabs[p].at[idx2[0]], gb[p], gsem[p]).wait()

        def wait_b():
            pltpu.make_async_copy(
                base_hbm.at[pl.ds(0, GR)], bbuf, bsem).wait()

        def group_code(t, i, g, buf):
            row0 = g * GR
            wrap = g == ngrp - 1
            gn = (g + 1) & (ngrp - 1)
            i_n = jnp.where(wrap, (t + 1) & (M - 1), i)
            not_last = jnp.logical_not(
                jnp.logical_and(t == nsweep - 1, wrap))
            build_idx(buf, gn * GR, i_n * (M * K))

            wait_b()
            wait_g(0)

            def n0_body(nn, r):
                mv, mi = inf16, zero16
                for sg in range(SGP):
                    a = bbuf[nn, pl.ds(sg * _L, _L)]
                    for j in range(M):
                        a = a + gb[0][j * GR + nn, pl.ds(sg * _L, _L)]
                    kv = sg * _L + iota
                    pr = a < mv
                    mv = jnp.where(pr, a, mv)
                    mi = jnp.where(pr, kv, mi)
                mv_s[nn, :] = mv
                mi_s[nn, :] = mi
                return r

            lax.fori_loop(0, GR, n0_body, 0)

            @pl.when(not_last)
            def _():
                issue_g(0, buf)

            wait_g(1)

            def n1_body(nn, r):
                mv = mv_s[nn, :]
                mi = mi_s[nn, :]
                for sg in range(SGP):
                    a = bbuf[nn, pl.ds(KP + sg * _L, _L)]
                    for j in range(M):
                        a = a + gb[1][j * GR + nn, pl.ds(sg * _L, _L)]
                    kv = KP + sg * _L + iota
                    pr = a < mv
                    mv = jnp.where(pr, a, mv)
                    mi = jnp.where(pr, kv, mi)
                for sh in (8, 4, 2, 1):
                    perm = iota ^ sh
                    mv2 = mv.at[perm].get(mode="promise_in_bounds")
                    mi2 = mi.at[perm].get(mode="promise_in_bounds")
                    tk = (mv2 < mv) | ((mv2 == mv) & (mi2 < mi))
                    mv = jnp.where(tk, mv2, mv)
                    mi = jnp.where(tk, mi2, mi)
                return jnp.where(iota == nn, mi, r)

            res = lax.fori_loop(0, GR, n1_body, zero16)

            @pl.when(not_last)
            def _():
                issue_g(1, buf)
                issue_b(gn * GR, i_n)

            b_loc[pl.ds(i * rows_per_w + row0, _L)] = res

        build_idx(idx2[1], 0, 0)
        issue_g(0, idx2[1])
        issue_g(1, idx2[1])
        issue_b(0, 0)

        def sweep_body(t, carry):
            i = t & (M - 1)

            def pair_body(q, carry2):
                group_code(t, i, 2 * q, idx2[0])
                group_code(t, i, 2 * q + 1, idx2[1])
                return carry2

            return lax.fori_loop(0, ngrp // 2, pair_body, carry)

        lax.fori_loop(0, nsweep, sweep_body, 0)
        for j in range(M):
            pltpu.sync_copy(b_loc.at[pl.ds(j * rows_per_w, rows_per_w)],
                            out_hbm.at[pl.ds(j * N + n0, rows_per_w)])

    return icm


def kernel(X, B, C_unary, C_pair, C):
    N, D = X.shape
    M = B.shape[1]
    K = C_pair.shape[2]

    base = _make_base(N, M, K, D, BN=512)(X, C, C_unary)
    tabs = _make_transpose(M * M, K)(C_pair.reshape(M * M, K, K))
    rows_per_w = N // _NW
    out = _make_icm(N, M, K, 2, rows_per_w)(
        base, *tabs, B.T.reshape(-1))
    return out.reshape(M, N).T


# speedup vs baseline: 1.3649x; 1.3649x over previous
import functools

import jax
import jax.numpy as jnp
from jax import lax
from jax.experimental import pallas as pl
from jax.experimental.pallas import tpu as pltpu
from jax.experimental.pallas import tpu_sc as plsc

_NC = 2
_NS = 16
_NW = _NC * _NS
_L = 16
_NPH = 4


def _make_base(N, M, K, D, BN):
    nb = N // BN

    def body(x_ref, c_ref, u_ref, o_ref):
        acc = lax.dot_general(x_ref[...], c_ref[...], (((1,), (1,)), ((), ())),
                              preferred_element_type=jnp.float32)
        o_ref[...] = -2.0 * acc + u_ref[...]

    return pl.pallas_call(
        body,
        grid=(M, nb),
        in_specs=[
            pl.BlockSpec((BN, D), lambda i, b: (b, 0)),
            pl.BlockSpec((K, D), lambda i, b: (i, 0)),
            pl.BlockSpec((BN, K), lambda i, b: (b, i)),
        ],
        out_specs=pl.BlockSpec((BN, K), lambda i, b, nb=nb: (i * nb + b, 0)),
        out_shape=jax.ShapeDtypeStruct((M * N, K), jnp.float32),
    )


def _make_transpose(MM, K):
    KP = K // _NPH

    def body(i_ref, *o_refs):
        x = i_ref[0]
        for p in range(_NPH):
            o_refs[p][...] = x[p * KP:(p + 1) * KP, :].T

    return pl.pallas_call(
        body,
        grid=(MM,),
        in_specs=[pl.BlockSpec((1, K, K), lambda g: (g, 0, 0))],
        out_specs=[pl.BlockSpec((K, KP), lambda g: (g, 0))
                   for _ in range(_NPH)],
        out_shape=[jax.ShapeDtypeStruct((MM * K, KP), jnp.float32)
                   for _ in range(_NPH)],
    )


def _make_icm(N, M, K, n_iter, rows_per_w):
    KP = K // _NPH
    GR = _L
    ngrp = rows_per_w // GR
    nidx = GR * M
    nsweep = n_iter * M
    mesh = plsc.VectorSubcoreMesh(core_axis_name="c", subcore_axis_name="s")

    @functools.partial(
        pl.kernel,
        mesh=mesh,
        out_type=jax.ShapeDtypeStruct((M * N,), jnp.int32),
        scratch_types=[
            pltpu.VMEM((M * rows_per_w,), jnp.int32),
            [pltpu.VMEM((nidx,), jnp.int32) for _ in range(2)],
            [pltpu.VMEM((nidx, KP), jnp.float32) for _ in range(_NPH)],
            [pltpu.VMEM((GR, K), jnp.float32) for _ in range(2)],
            pltpu.VMEM((GR, _L), jnp.float32),
            pltpu.VMEM((GR, _L), jnp.int32),
            [pltpu.SemaphoreType.DMA for _ in range(_NPH)],
            [pltpu.SemaphoreType.DMA for _ in range(2)],
        ],
    )
    def icm(base_hbm, t0, t1, t2, t3, b_hbm, out_hbm,
            b_loc, idx2, gb, bb, mv_s, mi_s, gsem, bsem):
        tabs = (t0, t1, t2, t3)
        wid = lax.axis_index("s") * _NC + lax.axis_index("c")
        n0 = wid * rows_per_w
        iota = lax.iota(jnp.int32, _L)
        inf16 = jnp.broadcast_to(jnp.float32(jnp.inf), (_L,))
        zero16 = jnp.broadcast_to(jnp.int32(0), (_L,))

        for j in range(M):
            pltpu.sync_copy(b_hbm.at[pl.ds(j * N + n0, rows_per_w)],
                            b_loc.at[pl.ds(j * rows_per_w, rows_per_w)])

        def build_idx(buf, row0, toff):
            for j in range(M):
                bv = b_loc[pl.ds(j * rows_per_w + row0, _L)]
                buf[pl.ds(j * _L, _L)] = bv + (toff + j * K)

        def issue_g(p, buf):
            pltpu.async_copy(tabs[p].at[buf], gb[p], gsem[p])

        def issue_b(bsel, row0, i_nxt):
            pltpu.async_copy(
                base_hbm.at[pl.ds(i_nxt * N + n0 + row0, GR)],
                bb[bsel], bsem[bsel])

        def wait_g(p):
            pltpu.make_async_copy(tabs[p].at[idx2[0]], gb[p], gsem[p]).wait()

        def wait_b(bsel):
            pltpu.make_async_copy(
                base_hbm.at[pl.ds(0, GR)], bb[bsel], bsem[bsel]).wait()

        def group_code(t, i, g, buf, bsel, res_carry):
            row0 = g * GR
            wrap = g == ngrp - 1
            gn = (g + 1) & (ngrp - 1)
            i_n = jnp.where(wrap, (t + 1) & (M - 1), i)
            not_last = jnp.logical_not(
                jnp.logical_and(t == nsweep - 1, wrap))
            build_idx(buf, gn * GR, i_n * (M * K))
            wait_b(bsel)

            res = res_carry
            for p in range(_NPH):
                wait_g(p)

                def n_body(nn, r, p=p):
                    if p == 0:
                        mv = inf16
                        mi = zero16
                    else:
                        mv = mv_s[nn, :]
                        mi = mi_s[nn, :]
                    for sg in range(KP // _L):
                        a = bb[bsel][nn, pl.ds(p * KP + sg * _L, _L)]
                        for j in range(M):
                            a = a + gb[p][j * GR + nn, pl.ds(sg * _L, _L)]
                        kv = (p * KP) + sg * _L + iota
                        pr = a < mv
                        mv = jnp.where(pr, a, mv)
                        mi = jnp.where(pr, kv, mi)
                    if p < _NPH - 1:
                        mv_s[nn, :] = mv
                        mi_s[nn, :] = mi
                        return r
                    for sh in (8, 4, 2, 1):
                        perm = iota ^ sh
                        mv2 = mv.at[perm].get(mode="promise_in_bounds")
                        mi2 = mi.at[perm].get(mode="promise_in_bounds")
                        tk = (mv2 < mv) | ((mv2 == mv) & (mi2 < mi))
                        mv = jnp.where(tk, mv2, mv)
                        mi = jnp.where(tk, mi2, mi)
                    return jnp.where(iota == nn, mi, r)

                res = lax.fori_loop(0, GR, n_body, res)

                @pl.when(not_last)
                def _(p=p):
                    issue_g(p, buf)
                    if p == 1:
                        issue_b(1 - bsel, gn * GR, i_n)

            b_loc[pl.ds(i * rows_per_w + row0, _L)] = res

        build_idx(idx2[1], 0, 0)
        for p in range(_NPH):
            issue_g(p, idx2[1])
        issue_b(0, 0, 0)

        def sweep_body(t, carry):
            i = t & (M - 1)

            def pair_body(q, carry2):
                group_code(t, i, 2 * q, idx2[0], 0, zero16)
                group_code(t, i, 2 * q + 1, idx2[1], 1, zero16)
                return carry2

            return lax.fori_loop(0, ngrp // 2, pair_body, carry)

        lax.fori_loop(0, nsweep, sweep_body, 0)
        for j in range(M):
            pltpu.sync_copy(b_loc.at[pl.ds(j * rows_per_w, rows_per_w)],
                            out_hbm.at[pl.ds(j * N + n0, rows_per_w)])

    return icm


def kernel(X, B, C_unary, C_pair, C):
    N, D = X.shape
    M = B.shape[1]
    K = C_pair.shape[2]

    base = _make_base(N, M, K, D, BN=512)(X, C, C_unary)
    tabs = _make_transpose(M * M, K)(C_pair.reshape(M * M, K, K))
    rows_per_w = N // _NW
    out = _make_icm(N, M, K, 2, rows_per_w)(
        base, *tabs, B.T.reshape(-1))
    return out.reshape(M, N).T
